# Initial kernel scaffold; baseline (speedup 1.0000x reference)
import jax
import jax.numpy as jnp
from jax.experimental import pallas as pl


def kernel(x, edge_index, edge_attr, params, weights):
    raise NotImplementedError("write your pallas kernel here")


# speedup vs baseline: 15.1209x; 15.1209x over previous
import functools
import math

import jax
import jax.numpy as jnp
from jax import lax
from jax.experimental import pallas as pl
from jax.experimental.pallas import tpu as pltpu
from jax.experimental.pallas import tpu_sc as plsc

N = 10000
E = 160000
F0 = 124
P = 4
H = 8
HC = 16
D = 16
ED = 16
OUT = 4
HF = H * HC

NPAD = 10240
ROW = HF + 16
TB = 1024
EB = 40
EPT = E // 32
NIT = EPT // EB
ZCH = 128
RPT = NPAD // 16


def _head_mat(inv=False):
    r = lax.broadcasted_iota(jnp.int32, (HF, H), 0) // HC
    c = lax.broadcasted_iota(jnp.int32, (HF, H), 1)
    m = (r == c).astype(jnp.float32)
    return m.T if inv else m


def _self_terms(xl, xr, ecl, attf):
    u = xl + xr + ecl
    w = jnp.maximum(u, 0.2 * u) * attf
    al = jnp.dot(w, _head_mat(), preferred_element_type=jnp.float32, precision=lax.Precision.HIGHEST)
    return jnp.exp(al)


def _prep_body(y_ref, wl_ref, wr_ref, ecl_ref, attf_ref,
               xl_ref, xr_ref, exs_ref):
    y = y_ref[...]
    xl = jnp.dot(y, wl_ref[...], preferred_element_type=jnp.float32)
    xr = jnp.dot(y, wr_ref[...], preferred_element_type=jnp.float32)
    xl_ref[...] = xl
    xr_ref[...] = xr
    exs_ref[...] = _self_terms(xl, xr, ecl_ref[...], attf_ref[...])


def _tc_prep(y, wlT, wrT, ecl, attf):
    grid = (NPAD // TB,)
    return pl.pallas_call(
        _prep_body,
        grid=grid,
        in_specs=[
            pl.BlockSpec((TB, HF), lambda i: (i, 0)),
            pl.BlockSpec((HF, HF), lambda i: (0, 0)),
            pl.BlockSpec((HF, HF), lambda i: (0, 0)),
            pl.BlockSpec((1, HF), lambda i: (0, 0)),
            pl.BlockSpec((1, HF), lambda i: (0, 0)),
        ],
        out_specs=[
            pl.BlockSpec((TB, HF), lambda i: (i, 0)),
            pl.BlockSpec((TB, HF), lambda i: (i, 0)),
            pl.BlockSpec((TB, H), lambda i: (i, 0)),
        ],
        out_shape=[
            jax.ShapeDtypeStruct((NPAD, HF), jnp.float32),
            jax.ShapeDtypeStruct((NPAD, HF), jnp.float32),
            jax.ShapeDtypeStruct((NPAD, H), jnp.float32),
        ],
    )(y, wlT, wrT, ecl, attf)


def _ec_body(ea_ref, weT_ref, ec_ref):
    ec_ref[...] = jnp.dot(ea_ref[...], weT_ref[...],
                          preferred_element_type=jnp.float32)


def _tc_ec(ea, weT):
    eb = 1600
    return pl.pallas_call(
        _ec_body,
        grid=(E // eb,),
        in_specs=[
            pl.BlockSpec((eb, ED), lambda i: (i, 0)),
            pl.BlockSpec((ED, HF), lambda i: (0, 0)),
        ],
        out_specs=pl.BlockSpec((eb, HF), lambda i: (i, 0)),
        out_shape=jax.ShapeDtypeStruct((E, HF), jnp.float32),
    )(ea, weT)


def _gelu(x):
    return 0.5 * x * (1.0 + lax.erf(x / math.sqrt(2.0)))


def _normalize(acc_ref, xl_ref, exs_ref):
    accT = acc_ref[0] + acc_ref[1]
    exs = exs_ref[...]
    gT = _head_mat(inv=True)
    exs128 = jnp.dot(exs, gT, preferred_element_type=jnp.float32, precision=lax.Precision.HIGHEST)
    num = accT[:, :HF] + exs128 * xl_ref[...]
    den = accT[:, HF:HF + H] + exs
    rec = 1.0 / (den + 1e-16)
    rec128 = jnp.dot(rec, gT, preferred_element_type=jnp.float32, precision=lax.Precision.HIGHEST)
    return num * rec128


def _combine_body(acc_ref, xl_ref, exs_ref, b_ref, wl_ref, wr_ref, attf_ref,
                  xl2_ref, xr2_ref, exs2_ref):
    out = _normalize(acc_ref, xl_ref, exs_ref) + b_ref[...]
    y2 = _gelu(out)
    xl2 = jnp.dot(y2, wl_ref[...], preferred_element_type=jnp.float32)
    xr2 = jnp.dot(y2, wr_ref[...], preferred_element_type=jnp.float32)
    xl2_ref[...] = xl2
    xr2_ref[...] = xr2
    zero = jnp.zeros((1, HF), jnp.float32)
    exs2_ref[...] = _self_terms(xl2, xr2, zero, attf_ref[...])


def _tc_combine(acc, xl, exs, b_row, wlT, wrT, attf):
    grid = (NPAD // TB,)
    return pl.pallas_call(
        _combine_body,
        grid=grid,
        in_specs=[
            pl.BlockSpec((2, TB, ROW), lambda i: (0, i, 0)),
            pl.BlockSpec((TB, HF), lambda i: (i, 0)),
            pl.BlockSpec((TB, H), lambda i: (i, 0)),
            pl.BlockSpec((1, HF), lambda i: (0, 0)),
            pl.BlockSpec((HF, HF), lambda i: (0, 0)),
            pl.BlockSpec((HF, HF), lambda i: (0, 0)),
            pl.BlockSpec((1, HF), lambda i: (0, 0)),
        ],
        out_specs=[
            pl.BlockSpec((TB, HF), lambda i: (i, 0)),
            pl.BlockSpec((TB, HF), lambda i: (i, 0)),
            pl.BlockSpec((TB, H), lambda i: (i, 0)),
        ],
        out_shape=[
            jax.ShapeDtypeStruct((NPAD, HF), jnp.float32),
            jax.ShapeDtypeStruct((NPAD, HF), jnp.float32),
            jax.ShapeDtypeStruct((NPAD, H), jnp.float32),
        ],
    )(acc, xl, exs, b_row, wlT, wrT, attf)


def _ln(t, g, b):
    m = jnp.mean(t, axis=-1, keepdims=True)
    v = jnp.mean((t - m) ** 2, axis=-1, keepdims=True)
    return (t - m) / jnp.sqrt(v + 1e-5) * g + b


def _final_body(acc_ref, xl_ref, exs_ref, a16_ref, w1_ref, w2_ref, b1_ref,
                vv_ref, cw_ref, out_ref):
    outc = _normalize(acc_ref, xl_ref, exs_ref)
    r = lax.broadcasted_iota(jnp.int32, (HF, HC), 0) % HC
    c = lax.broadcasted_iota(jnp.int32, (HF, HC), 1)
    hm = (r == c).astype(jnp.float32) * (1.0 / H)
    a16 = a16_ref[...]
    vv = vv_ref[...]
    y4 = jnp.dot(outc, hm, preferred_element_type=jnp.float32, precision=lax.Precision.HIGHEST) + vv[22:23, :]
    t = _gelu(y4)
    mem = t
    for l in range(2):
        mb = a16[l * 64:(l + 1) * 64]
        vb = vv[l * 11:(l + 1) * 11]
        sa_wv, sa_wo = mb[0:16], mb[16:32]
        ca_wv, ca_wo = mb[32:48], mb[48:64]
        o = jnp.dot(jnp.dot(t, sa_wv, preferred_element_type=jnp.float32)
                    + vb[0:1], sa_wo,
                    preferred_element_type=jnp.float32) + vb[1:2]
        t = _ln(t + o, vb[5:6], vb[6:7])
        o2 = jnp.dot(jnp.dot(mem, ca_wv, preferred_element_type=jnp.float32)
                     + vb[2:3], ca_wo,
                     preferred_element_type=jnp.float32) + vb[3:4]
        t = _ln(t + o2, vb[7:8], vb[8:9])
        h = jax.nn.relu(jnp.dot(t, w1_ref[l * 16:(l + 1) * 16],
                                preferred_element_type=jnp.float32)
                        + b1_ref[l:l + 1])
        ff = jnp.dot(h, w2_ref[l * 128:(l + 1) * 128],
                     preferred_element_type=jnp.float32) + vb[4:5]
        t = _ln(t + ff, vb[9:10], vb[10:11])
    t = jnp.tanh(t)
    out_ref[...] = jnp.dot(t, cw_ref[...],
                           preferred_element_type=jnp.float32) + vv[23:24, 0:OUT]


def _tc_final(acc, xl, exs, a16, w1s, w2s, b1s, vv, cwT):
    grid = (NPAD // TB,)
    return pl.pallas_call(
        _final_body,
        grid=grid,
        in_specs=[
            pl.BlockSpec((2, TB, ROW), lambda i: (0, i, 0)),
            pl.BlockSpec((TB, HF), lambda i: (i, 0)),
            pl.BlockSpec((TB, H), lambda i: (i, 0)),
            pl.BlockSpec((128, 16), lambda i: (0, 0)),
            pl.BlockSpec((32, 128), lambda i: (0, 0)),
            pl.BlockSpec((256, 16), lambda i: (0, 0)),
            pl.BlockSpec((2, 128), lambda i: (0, 0)),
            pl.BlockSpec((24, 16), lambda i: (0, 0)),
            pl.BlockSpec((16, OUT), lambda i: (0, 0)),
        ],
        out_specs=pl.BlockSpec((TB, OUT), lambda i: (i, 0)),
        out_shape=jax.ShapeDtypeStruct((NPAD, OUT), jnp.float32),
    )(acc, xl, exs, a16, w1s, w2s, b1s, vv, cwT)


def _sc_edges_body(use_ec, xl_hbm, xr_hbm, ec_hbm, src_hbm, dst_hbm, att_hbm,
                   zeros_hbm, out_hbm, acc_sh, idx_s, idx_d, gl, gr, gec,
                   sbuf, att_v, sem0, sem1):
    cid = lax.axis_index("c")
    sid = lax.axis_index("s")
    wid = sid * 2 + cid

    for k in range(RPT // ZCH):
        pltpu.sync_copy(zeros_hbm, acc_sh.at[pl.ds(sid * RPT + k * ZCH, ZCH)])
    pltpu.sync_copy(att_hbm, att_v)
    plsc.subcore_barrier()

    iota16 = lax.iota(jnp.int32, 16)
    perms = [iota16 ^ k for k in (1, 2, 4, 8)]
    ebase = wid * EPT

    def batch(i, _):
        eb = ebase + i * EB
        pltpu.sync_copy(src_hbm.at[pl.ds(eb, EB)], idx_s)
        pltpu.sync_copy(dst_hbm.at[pl.ds(eb, EB)], idx_d)
        pltpu.async_copy(xl_hbm.at[idx_s], gl, sem0).wait()
        pltpu.async_copy(xr_hbm.at[idx_d], gr, sem1).wait()
        if use_ec:
            pltpu.sync_copy(ec_hbm.at[pl.ds(eb, EB)], gec)

        def edge(e, _):
            den_v = jnp.zeros((16,), jnp.float32)
            for h in range(H):
                a = gl[e, pl.ds(h * HC, 16)]
                u = a + gr[e, pl.ds(h * HC, 16)]
                if use_ec:
                    u = u + gec[e, pl.ds(h * HC, 16)]
                w = jnp.maximum(u, 0.2 * u) * att_v[h, :]
                for pm in perms:
                    w = w + w.at[pm].get(mode='promise_in_bounds')
                exv = jnp.exp(w)
                sbuf[e, pl.ds(h * HC, 16)] = a * exv
                den_v = jnp.where(iota16 == h, exv, den_v)
            sbuf[e, pl.ds(HF, 16)] = den_v
            return 0

        lax.fori_loop(0, EB, edge, 0)
        pltpu.sync_copy(sbuf, acc_sh.at[idx_d], add=True)
        return 0

    lax.fori_loop(0, NIT, batch, 0)
    plsc.subcore_barrier()

    for k in range(RPT // ZCH):
        r0 = sid * RPT + k * ZCH
        pltpu.sync_copy(acc_sh.at[pl.ds(r0, ZCH)],
                        out_hbm.at[cid, pl.ds(r0, ZCH)])


def _sc_edges(xl, xr, ec, src, dst, att, zeros_blk, use_ec):
    mesh = plsc.VectorSubcoreMesh(core_axis_name="c", subcore_axis_name="s")
    kfn = pl.kernel(
        functools.partial(_sc_edges_body, use_ec),
        mesh=mesh,
        compiler_params=pltpu.CompilerParams(use_tc_tiling_on_sc=False),
        out_type=jax.ShapeDtypeStruct((2, NPAD, ROW), jnp.float32),
        scratch_types=[
            pltpu.VMEM_SHARED((NPAD, ROW), jnp.float32),
            pltpu.VMEM((EB,), jnp.int32),
            pltpu.VMEM((EB,), jnp.int32),
            pltpu.VMEM((EB, HF), jnp.float32),
            pltpu.VMEM((EB, HF), jnp.float32),
            pltpu.VMEM((EB, HF), jnp.float32),
            pltpu.VMEM((EB, ROW), jnp.float32),
            pltpu.VMEM((H, HC), jnp.float32),
            pltpu.SemaphoreType.DMA,
            pltpu.SemaphoreType.DMA,
        ],
    )
    return kfn(xl, xr, ec, src, dst, att, zeros_blk)


def kernel(x, edge_index, edge_attr, params, weights):
    g1, g2, g3 = weights['g1'], weights['g2'], weights['g3']
    dec, conv = weights['dec'], weights['conv']

    xp = x.reshape(N, F0)
    pb = jnp.broadcast_to(params.reshape(1, P), (N, P))
    y = jnp.concatenate([xp, pb], axis=1)
    y = jnp.pad(y, ((0, NPAD - N), (0, 0)))

    src = edge_index[0]
    dst = edge_index[1]

    ecl1 = (jnp.mean(edge_attr, axis=0, keepdims=True) @ g1['We'].T)
    zrow = jnp.zeros((1, HF), jnp.float32)
    zeros_blk = jnp.zeros((ZCH, ROW), jnp.float32)

    attf1 = g1['att'].reshape(1, HF)
    attf2 = g2['att'].reshape(1, HF)
    attf3 = g3['att'].reshape(1, HF)

    ec = _tc_ec(edge_attr, g1['We'].T)
    xl1, xr1, exs1 = _tc_prep(y, g1['Wl'].T, g1['Wr'].T, ecl1, attf1)
    acc1 = _sc_edges(xl1, xr1, ec, src, dst, g1['att'], zeros_blk, True)

    xl2, xr2, exs2 = _tc_combine(acc1, xl1, exs1, g1['b'].reshape(1, HF),
                                 g2['Wl'].T, g2['Wr'].T, attf2)
    acc2 = _sc_edges(xl2, xr2, ec, src, dst, g2['att'], zeros_blk, False)

    xl3, xr3, exs3 = _tc_combine(acc2, xl2, exs2, g2['b'].reshape(1, HF),
                                 g3['Wl'].T, g3['Wr'].T, attf3)
    acc3 = _sc_edges(xl3, xr3, ec, src, dst, g3['att'], zeros_blk, False)

    a16 = jnp.concatenate([
        jnp.concatenate([L['sa_Wv'].T, L['sa_Wo'].T, L['ca_Wv'].T,
                         L['ca_Wo'].T], axis=0) for L in dec], axis=0)
    w1s = jnp.concatenate([dec[0]['W1'].T, dec[1]['W1'].T], axis=0)
    w2s = jnp.concatenate([dec[0]['W2'].T, dec[1]['W2'].T], axis=0)
    b1s = jnp.stack([dec[0]['b1'], dec[1]['b1']], axis=0)
    vrows = []
    for L in dec:
        vrows += [L['sa_bv'], L['sa_bo'], L['ca_bv'], L['ca_bo'], L['b2'],
                  L['ln1_g'], L['ln1_b'], L['ln2_g'], L['ln2_b'],
                  L['ln3_g'], L['ln3_b']]
    vrows.append(g3['b'])
    vrows.append(jnp.pad(conv['b'], (0, HC - OUT)))
    vv = jnp.stack(vrows, axis=0)

    out = _tc_final(acc3, xl3, exs3, a16, w1s, w2s, b1s, vv, conv['W'].T)
    return out[:N].reshape(1, N, OUT)


# speedup vs baseline: 45.2095x; 2.9899x over previous
import functools
import math

import jax
import jax.numpy as jnp
from jax import lax
from jax.experimental import pallas as pl
from jax.experimental.pallas import tpu as pltpu
from jax.experimental.pallas import tpu_sc as plsc

N = 10000
E = 160000
F0 = 124
P = 4
H = 8
HC = 16
D = 16
ED = 16
OUT = 4
HF = H * HC

NPAD = 10240
ROW = HF + 16
TB = 1024
EB = 40
EPT = E // 32
NIT = EPT // EB
ZCH = 128
RPT = NPAD // 16


def _head_mat(inv=False):
    r = lax.broadcasted_iota(jnp.int32, (HF, H), 0) // HC
    c = lax.broadcasted_iota(jnp.int32, (HF, H), 1)
    m = (r == c).astype(jnp.float32)
    return m.T if inv else m


def _self_terms(xl, xr, ecl, attf):
    u = xl + xr + ecl
    w = jnp.maximum(u, 0.2 * u) * attf
    al = jnp.dot(w, _head_mat(), preferred_element_type=jnp.float32, precision=lax.Precision.HIGHEST)
    return jnp.exp(al)


def _prep_body(y_ref, wl_ref, wr_ref, ecl_ref, attf_ref,
               xl_ref, xr_ref, exs_ref):
    y = y_ref[...]
    xl = jnp.dot(y, wl_ref[...], preferred_element_type=jnp.float32)
    xr = jnp.dot(y, wr_ref[...], preferred_element_type=jnp.float32)
    xl_ref[...] = xl
    xr_ref[...] = xr
    exs_ref[...] = _self_terms(xl, xr, ecl_ref[...], attf_ref[...])


def _tc_prep(y, wlT, wrT, ecl, attf):
    grid = (NPAD // TB,)
    return pl.pallas_call(
        _prep_body,
        grid=grid,
        in_specs=[
            pl.BlockSpec((TB, HF), lambda i: (i, 0)),
            pl.BlockSpec((HF, HF), lambda i: (0, 0)),
            pl.BlockSpec((HF, HF), lambda i: (0, 0)),
            pl.BlockSpec((1, HF), lambda i: (0, 0)),
            pl.BlockSpec((1, HF), lambda i: (0, 0)),
        ],
        out_specs=[
            pl.BlockSpec((TB, HF), lambda i: (i, 0)),
            pl.BlockSpec((TB, HF), lambda i: (i, 0)),
            pl.BlockSpec((TB, H), lambda i: (i, 0)),
        ],
        out_shape=[
            jax.ShapeDtypeStruct((NPAD, HF), jnp.float32),
            jax.ShapeDtypeStruct((NPAD, HF), jnp.float32),
            jax.ShapeDtypeStruct((NPAD, H), jnp.float32),
        ],
    )(y, wlT, wrT, ecl, attf)


def _ec_body(ea_ref, weT_ref, ec_ref):
    ec_ref[...] = jnp.dot(ea_ref[...], weT_ref[...],
                          preferred_element_type=jnp.float32)


def _tc_ec(ea, weT):
    eb = 1600
    return pl.pallas_call(
        _ec_body,
        grid=(E // eb,),
        in_specs=[
            pl.BlockSpec((eb, ED), lambda i: (i, 0)),
            pl.BlockSpec((ED, HF), lambda i: (0, 0)),
        ],
        out_specs=pl.BlockSpec((eb, HF), lambda i: (i, 0)),
        out_shape=jax.ShapeDtypeStruct((E, HF), jnp.float32),
    )(ea, weT)


def _gelu(x):
    return 0.5 * x * (1.0 + lax.erf(x / math.sqrt(2.0)))


def _normalize(acc_ref, xl_ref, exs_ref):
    accT = acc_ref[0] + acc_ref[1]
    exs = exs_ref[...]
    gT = _head_mat(inv=True)
    exs128 = jnp.dot(exs, gT, preferred_element_type=jnp.float32, precision=lax.Precision.HIGHEST)
    num = accT[:, :HF] + exs128 * xl_ref[...]
    den = accT[:, HF:HF + H] + exs
    rec = 1.0 / (den + 1e-16)
    rec128 = jnp.dot(rec, gT, preferred_element_type=jnp.float32, precision=lax.Precision.HIGHEST)
    return num * rec128


def _combine_body(acc_ref, xl_ref, exs_ref, b_ref, wl_ref, wr_ref, attf_ref,
                  xl2_ref, xr2_ref, exs2_ref):
    out = _normalize(acc_ref, xl_ref, exs_ref) + b_ref[...]
    y2 = _gelu(out)
    xl2 = jnp.dot(y2, wl_ref[...], preferred_element_type=jnp.float32)
    xr2 = jnp.dot(y2, wr_ref[...], preferred_element_type=jnp.float32)
    xl2_ref[...] = xl2
    xr2_ref[...] = xr2
    zero = jnp.zeros((1, HF), jnp.float32)
    exs2_ref[...] = _self_terms(xl2, xr2, zero, attf_ref[...])


def _tc_combine(acc, xl, exs, b_row, wlT, wrT, attf):
    grid = (NPAD // TB,)
    return pl.pallas_call(
        _combine_body,
        grid=grid,
        in_specs=[
            pl.BlockSpec((2, TB, ROW), lambda i: (0, i, 0)),
            pl.BlockSpec((TB, HF), lambda i: (i, 0)),
            pl.BlockSpec((TB, H), lambda i: (i, 0)),
            pl.BlockSpec((1, HF), lambda i: (0, 0)),
            pl.BlockSpec((HF, HF), lambda i: (0, 0)),
            pl.BlockSpec((HF, HF), lambda i: (0, 0)),
            pl.BlockSpec((1, HF), lambda i: (0, 0)),
        ],
        out_specs=[
            pl.BlockSpec((TB, HF), lambda i: (i, 0)),
            pl.BlockSpec((TB, HF), lambda i: (i, 0)),
            pl.BlockSpec((TB, H), lambda i: (i, 0)),
        ],
        out_shape=[
            jax.ShapeDtypeStruct((NPAD, HF), jnp.float32),
            jax.ShapeDtypeStruct((NPAD, HF), jnp.float32),
            jax.ShapeDtypeStruct((NPAD, H), jnp.float32),
        ],
    )(acc, xl, exs, b_row, wlT, wrT, attf)


def _ln(t, g, b):
    m = jnp.mean(t, axis=-1, keepdims=True)
    v = jnp.mean((t - m) ** 2, axis=-1, keepdims=True)
    return (t - m) / jnp.sqrt(v + 1e-5) * g + b


def _final_body(acc_ref, xl_ref, exs_ref, a16_ref, w1_ref, w2_ref, b1_ref,
                vv_ref, cw_ref, out_ref):
    outc = _normalize(acc_ref, xl_ref, exs_ref)
    r = lax.broadcasted_iota(jnp.int32, (HF, HC), 0) % HC
    c = lax.broadcasted_iota(jnp.int32, (HF, HC), 1)
    hm = (r == c).astype(jnp.float32) * (1.0 / H)
    a16 = a16_ref[...]
    vv = vv_ref[...]
    y4 = jnp.dot(outc, hm, preferred_element_type=jnp.float32, precision=lax.Precision.HIGHEST) + vv[22:23, :]
    t = _gelu(y4)
    mem = t
    for l in range(2):
        mb = a16[l * 64:(l + 1) * 64]
        vb = vv[l * 11:(l + 1) * 11]
        sa_wv, sa_wo = mb[0:16], mb[16:32]
        ca_wv, ca_wo = mb[32:48], mb[48:64]
        o = jnp.dot(jnp.dot(t, sa_wv, preferred_element_type=jnp.float32)
                    + vb[0:1], sa_wo,
                    preferred_element_type=jnp.float32) + vb[1:2]
        t = _ln(t + o, vb[5:6], vb[6:7])
        o2 = jnp.dot(jnp.dot(mem, ca_wv, preferred_element_type=jnp.float32)
                     + vb[2:3], ca_wo,
                     preferred_element_type=jnp.float32) + vb[3:4]
        t = _ln(t + o2, vb[7:8], vb[8:9])
        h = jax.nn.relu(jnp.dot(t, w1_ref[l * 16:(l + 1) * 16],
                                preferred_element_type=jnp.float32)
                        + b1_ref[l:l + 1])
        ff = jnp.dot(h, w2_ref[l * 128:(l + 1) * 128],
                     preferred_element_type=jnp.float32) + vb[4:5]
        t = _ln(t + ff, vb[9:10], vb[10:11])
    t = jnp.tanh(t)
    out_ref[...] = jnp.dot(t, cw_ref[...],
                           preferred_element_type=jnp.float32) + vv[23:24, 0:OUT]


def _tc_final(acc, xl, exs, a16, w1s, w2s, b1s, vv, cwT):
    grid = (NPAD // TB,)
    return pl.pallas_call(
        _final_body,
        grid=grid,
        in_specs=[
            pl.BlockSpec((2, TB, ROW), lambda i: (0, i, 0)),
            pl.BlockSpec((TB, HF), lambda i: (i, 0)),
            pl.BlockSpec((TB, H), lambda i: (i, 0)),
            pl.BlockSpec((128, 16), lambda i: (0, 0)),
            pl.BlockSpec((32, 128), lambda i: (0, 0)),
            pl.BlockSpec((256, 16), lambda i: (0, 0)),
            pl.BlockSpec((2, 128), lambda i: (0, 0)),
            pl.BlockSpec((24, 16), lambda i: (0, 0)),
            pl.BlockSpec((16, OUT), lambda i: (0, 0)),
        ],
        out_specs=pl.BlockSpec((TB, OUT), lambda i: (i, 0)),
        out_shape=jax.ShapeDtypeStruct((NPAD, OUT), jnp.float32),
    )(acc, xl, exs, a16, w1s, w2s, b1s, vv, cwT)


def _sc_edges_body(use_ec, xl_hbm, xr_hbm, ec_hbm, src_hbm, dst_hbm, att_hbm,
                   zeros_hbm, out_hbm, acc_sh, att_v, sbuf, *bufs):
    NBUF = 2 if use_ec else 3
    IS = bufs[0:NBUF]
    ID = bufs[NBUF:2 * NBUF]
    GL = bufs[2 * NBUF:3 * NBUF]
    GR = bufs[3 * NBUF:4 * NBUF]
    GE = bufs[4 * NBUF:5 * NBUF] if use_ec else [None] * NBUF
    SM = bufs[-NBUF:]

    cid = lax.axis_index("c")
    sid = lax.axis_index("s")
    wid = sid * 2 + cid

    for k in range(RPT // ZCH):
        pltpu.sync_copy(zeros_hbm, acc_sh.at[pl.ds(sid * RPT + k * ZCH, ZCH)])
    pltpu.sync_copy(att_hbm, att_v)
    plsc.subcore_barrier()

    iota16 = lax.iota(jnp.int32, 16)
    perms = [iota16 ^ k for k in (1, 2, 4, 8)]
    attv = [att_v[h, :] for h in range(H)]
    ebase = wid * EPT

    def start(i, k):
        eb = ebase + i * EB
        pltpu.sync_copy(src_hbm.at[pl.ds(eb, EB)], IS[k])
        pltpu.sync_copy(dst_hbm.at[pl.ds(eb, EB)], ID[k])
        hs = [pltpu.async_copy(xl_hbm.at[IS[k]], GL[k], SM[k]),
              pltpu.async_copy(xr_hbm.at[ID[k]], GR[k], SM[k])]
        if use_ec:
            hs.append(pltpu.async_copy(ec_hbm.at[pl.ds(eb, EB)], GE[k], SM[k]))
        return hs

    def finish(hs, k):
        for h in hs:
            h.wait()
        gl, gr, ge = GL[k], GR[k], GE[k]

        @plsc.parallel_loop(0, EB, unroll=4)
        def edge(e):
            den_v = jnp.zeros((16,), jnp.float32)
            for h in range(H):
                a = gl[e, pl.ds(h * HC, 16)]
                u = a + gr[e, pl.ds(h * HC, 16)]
                if use_ec:
                    u = u + ge[e, pl.ds(h * HC, 16)]
                w = jnp.maximum(u, 0.2 * u) * attv[h]
                for pm in perms:
                    w = w + w.at[pm].get(mode='promise_in_bounds')
                exv = jnp.exp(w)
                sbuf[e, pl.ds(h * HC, 16)] = a * exv
                den_v = jnp.where(iota16 == h, exv, den_v)
            sbuf[e, pl.ds(HF, 16)] = den_v

        pltpu.sync_copy(sbuf, acc_sh.at[ID[k]], add=True)

    def quad(q, _):
        b0 = q * NBUF
        hss = [start(b0 + k, k) for k in range(NBUF)]
        for k in range(NBUF):
            finish(hss[k], k)
        return 0

    lax.fori_loop(0, NIT // NBUF, quad, 0)
    for t in range(NIT % NBUF):
        finish(start(NIT - (NIT % NBUF) + t, t), t)
    plsc.subcore_barrier()

    for k in range(RPT // ZCH):
        r0 = sid * RPT + k * ZCH
        pltpu.sync_copy(acc_sh.at[pl.ds(r0, ZCH)],
                        out_hbm.at[cid, pl.ds(r0, ZCH)])


def _sc_edges(xl, xr, ec, src, dst, att, zeros_blk, use_ec):
    nbuf = 2 if use_ec else 3
    mesh = plsc.VectorSubcoreMesh(core_axis_name="c", subcore_axis_name="s")
    kfn = pl.kernel(
        functools.partial(_sc_edges_body, use_ec),
        mesh=mesh,
        compiler_params=pltpu.CompilerParams(use_tc_tiling_on_sc=False),
        out_type=jax.ShapeDtypeStruct((2, NPAD, ROW), jnp.float32),
        scratch_types=(
            [pltpu.VMEM_SHARED((NPAD, ROW), jnp.float32),
             pltpu.VMEM((H, HC), jnp.float32),
             pltpu.VMEM((EB, ROW), jnp.float32)]
            + [pltpu.VMEM((EB,), jnp.int32) for _ in range(2 * nbuf)]
            + [pltpu.VMEM((EB, HF), jnp.float32)
               for _ in range((3 if use_ec else 2) * nbuf)]
            + [pltpu.SemaphoreType.DMA for _ in range(nbuf)]
        ),
    )
    return kfn(xl, xr, ec, src, dst, att, zeros_blk)


def kernel(x, edge_index, edge_attr, params, weights):
    g1, g2, g3 = weights['g1'], weights['g2'], weights['g3']
    dec, conv = weights['dec'], weights['conv']

    xp = x.reshape(N, F0)
    pb = jnp.broadcast_to(params.reshape(1, P), (N, P))
    y = jnp.concatenate([xp, pb], axis=1)
    y = jnp.pad(y, ((0, NPAD - N), (0, 0)))

    src = edge_index[0]
    dst = edge_index[1]

    ecl1 = (jnp.mean(edge_attr, axis=0, keepdims=True) @ g1['We'].T)
    zrow = jnp.zeros((1, HF), jnp.float32)
    zeros_blk = jnp.zeros((ZCH, ROW), jnp.float32)

    attf1 = g1['att'].reshape(1, HF)
    attf2 = g2['att'].reshape(1, HF)
    attf3 = g3['att'].reshape(1, HF)

    ec = _tc_ec(edge_attr, g1['We'].T)
    xl1, xr1, exs1 = _tc_prep(y, g1['Wl'].T, g1['Wr'].T, ecl1, attf1)
    acc1 = _sc_edges(xl1, xr1, ec, src, dst, g1['att'], zeros_blk, True)

    xl2, xr2, exs2 = _tc_combine(acc1, xl1, exs1, g1['b'].reshape(1, HF),
                                 g2['Wl'].T, g2['Wr'].T, attf2)
    acc2 = _sc_edges(xl2, xr2, ec, src, dst, g2['att'], zeros_blk, False)

    xl3, xr3, exs3 = _tc_combine(acc2, xl2, exs2, g2['b'].reshape(1, HF),
                                 g3['Wl'].T, g3['Wr'].T, attf3)
    acc3 = _sc_edges(xl3, xr3, ec, src, dst, g3['att'], zeros_blk, False)

    a16 = jnp.concatenate([
        jnp.concatenate([L['sa_Wv'].T, L['sa_Wo'].T, L['ca_Wv'].T,
                         L['ca_Wo'].T], axis=0) for L in dec], axis=0)
    w1s = jnp.concatenate([dec[0]['W1'].T, dec[1]['W1'].T], axis=0)
    w2s = jnp.concatenate([dec[0]['W2'].T, dec[1]['W2'].T], axis=0)
    b1s = jnp.stack([dec[0]['b1'], dec[1]['b1']], axis=0)
    vrows = []
    for L in dec:
        vrows += [L['sa_bv'], L['sa_bo'], L['ca_bv'], L['ca_bo'], L['b2'],
                  L['ln1_g'], L['ln1_b'], L['ln2_g'], L['ln2_b'],
                  L['ln3_g'], L['ln3_b']]
    vrows.append(g3['b'])
    vrows.append(jnp.pad(conv['b'], (0, HC - OUT)))
    vv = jnp.stack(vrows, axis=0)

    out = _tc_final(acc3, xl3, exs3, a16, w1s, w2s, b1s, vv, conv['W'].T)
    return out[:N].reshape(1, N, OUT)
